# baseline (speedup 1.0000x reference)
import jax
import jax.numpy as jnp
from jax import lax
from jax.experimental import pallas as pl
from jax.experimental.pallas import tpu as pltpu
from jax.experimental.pallas import tpu_sc as plsc

_NC, _NS = 2, 16
_NW = _NC * _NS

_G = 16384
_E = 26
_D = 32
_V = 1000000
_GPW = _G // _NW
_EH = _E // 2
_LH = 64
_CROWS = _EH * _LH
_NCHUNK = 16

_TCC = 16384


def _detile_body(x_ref, o_ref):
    y = x_ref[...].T
    o_ref[...] = jnp.concatenate(
        [lax.slice(y, (j, 0), (_TCC, _D), (4, 1)) for j in range(4)], axis=1)


def _chunk_coords(c):
    tc_l = c >> 2
    e0 = ((c >> 1) & 1) * _EH
    l0 = (c & 1) * _LH
    return tc_l, e0, l0


def _gather_body(idx_hbm, table_hbm, out_hbm,
                 idx_blk, idx_list, gbuf, out_stage, sem_g, sem_w):
    wid = lax.axis_index("s") * _NC + lax.axis_index("c")
    g0 = wid * _GPW
    iota = lax.iota(jnp.int32, 16)

    pltpu.sync_copy(idx_hbm.at[pl.ds(g0, _GPW), :], idx_blk)

    def build_idx(c, b):
        tc_l, e0, l0 = _chunk_coords(c)

        @plsc.parallel_loop(0, _EH, unroll=4)
        def _b(e_l):
            col = jnp.full((16,), e0 + e_l, jnp.int32)
            for lb in range(_LH // 16):
                rows = tc_l * 128 + l0 + lb * 16 + iota
                v = plsc.load_gather(idx_blk, [rows, col])
                idx_list[b][pl.ds(e_l * _LH + lb * 16, 16)] = v

    def start_gather(c, b):
        pltpu.async_copy(table_hbm.at[idx_list[b]], gbuf[b], sem_g[b])

    def wait_gather(b):
        pltpu.make_async_copy(table_hbm.at[idx_list[b]], gbuf[b],
                              sem_g[b]).wait()

    def transpose(b):
        @plsc.parallel_loop(0, _EH * _D, unroll=8)
        def _t(q):
            e_l = q >> 5
            f = q & 31
            colf = jnp.full((16,), f, jnp.int32)
            for lb in range(_LH // 16):
                rows = e_l * _LH + lb * 16 + iota
                v = plsc.load_gather(gbuf[b], [rows, colf])
                out_stage[b][e_l, f >> 3, 0, f & 7, pl.ds(lb * 16, 16)] = v

    def fire_writes(c, b):
        tc_l, e0, l0 = _chunk_coords(c)

        @pl.loop(0, _EH)
        def _w(e_l):
            pltpu.async_copy(
                out_stage[b].at[pl.ds(e_l, 1)],
                out_hbm.at[pl.ds(e0 + e_l, 1), :,
                           pl.ds(wid * 4 + tc_l, 1), :, pl.ds(l0, _LH)],
                sem_w[b])

    def drain_writes(c, b):
        tc_l, e0, l0 = _chunk_coords(c)

        @pl.loop(0, _EH)
        def _d(e_l):
            pltpu.make_async_copy(
                out_stage[b].at[pl.ds(e_l, 1)],
                out_hbm.at[pl.ds(e0 + e_l, 1), :,
                           pl.ds(wid * 4 + tc_l, 1), :, pl.ds(l0, _LH)],
                sem_w[b]).wait()

    build_idx(0, 0)
    start_gather(0, 0)

    @pl.loop(0, _NCHUNK, step=2)
    def _steps(c):
        for b in (0, 1):
            cc = c + b
            nxt = cc + 1
            o = b ^ 1

            @pl.when(nxt < _NCHUNK)
            def _():
                build_idx(nxt, o)

            wait_gather(b)

            @pl.when(nxt < _NCHUNK)
            def _():
                @pl.when(cc > 0)
                def _():
                    drain_writes(cc - 1, o)
                start_gather(nxt, o)

            transpose(b)
            fire_writes(cc, b)

    drain_writes(_NCHUNK - 2, 0)
    drain_writes(_NCHUNK - 1, 1)


@jax.jit
def kernel(indices, embed):
    table = embed

    mesh = plsc.VectorSubcoreMesh(core_axis_name="c", subcore_axis_name="s")
    out5 = pl.kernel(
        _gather_body,
        out_type=jax.ShapeDtypeStruct((_E, _D // 8, _G // 128, 8, 128),
                                      jnp.float32),
        mesh=mesh,
        scratch_types=[
            pltpu.VMEM((_GPW, _E), jnp.int32),
            [pltpu.VMEM((_CROWS,), jnp.int32) for _ in range(2)],
            [pltpu.VMEM((_CROWS, _D), jnp.float32) for _ in range(2)],
            [pltpu.VMEM((_EH, _D // 8, 1, 8, _LH), jnp.float32)
             for _ in range(2)],
            [pltpu.SemaphoreType.DMA for _ in range(2)],
            [pltpu.SemaphoreType.DMA for _ in range(2)],
        ],
        compiler_params=pltpu.CompilerParams(use_tc_tiling_on_sc=False,
                                             needs_layout_passes=False),
    )(indices.astype(jnp.int32), table)
    return out5.transpose(2, 4, 0, 1, 3).reshape(_G, _E, _D)


# speedup vs baseline: 1.0011x; 1.0011x over previous
import jax
import jax.numpy as jnp
from jax import lax
from jax.experimental import pallas as pl
from jax.experimental.pallas import tpu as pltpu
from jax.experimental.pallas import tpu_sc as plsc

_NC, _NS = 2, 16
_NW = _NC * _NS

_G = 16384
_E = 26
_D = 32
_V = 1000000
_GPW = _G // _NW
_EH = _E // 2
_LH = 64
_CROWS = _EH * _LH
_NCHUNK = 16


def _chunk_coords(c):
    tc_l = c >> 2
    e0 = ((c >> 1) & 1) * _EH
    l0 = (c & 1) * _LH
    return tc_l, e0, l0


def _gather_body(idx_hbm, table_hbm, out_hbm,
                 idx_blk, idx_list, gbuf, out_stage, sem_g, sem_w):
    wid = lax.axis_index("s") * _NC + lax.axis_index("c")
    g0 = wid * _GPW
    iota = lax.iota(jnp.int32, 16)

    pltpu.sync_copy(idx_hbm.at[pl.ds(g0, _GPW), :], idx_blk)

    def build_idx(c, b):
        tc_l, e0, l0 = _chunk_coords(c)

        @plsc.parallel_loop(0, _EH, unroll=4)
        def _b(e_l):
            col = jnp.full((16,), e0 + e_l, jnp.int32)
            for lb in range(_LH // 16):
                rows = tc_l * 128 + l0 + lb * 16 + iota
                v = plsc.load_gather(idx_blk, [rows, col])
                idx_list[b][pl.ds(e_l * _LH + lb * 16, 16)] = v

    def start_gather(c, b):
        pltpu.async_copy(table_hbm.at[idx_list[b]], gbuf[b], sem_g[b])

    def wait_gather(b):
        pltpu.make_async_copy(table_hbm.at[idx_list[b]], gbuf[b],
                              sem_g[b]).wait()

    def transpose(b):
        @plsc.parallel_loop(0, _EH * _D, unroll=8)
        def _t(q):
            e_l = q >> 5
            f = q & 31
            colf = jnp.full((16,), f, jnp.int32)
            for lb in range(_LH // 16):
                rows = e_l * _LH + lb * 16 + iota
                v = plsc.load_gather(gbuf[b], [rows, colf])
                out_stage[b][e_l, f >> 3, 0, f & 7, pl.ds(lb * 16, 16)] = v

    def fire_writes(c, b):
        tc_l, e0, l0 = _chunk_coords(c)

        @pl.loop(0, _EH)
        def _w(e_l):
            pltpu.async_copy(
                out_stage[b].at[pl.ds(e_l, 1)],
                out_hbm.at[pl.ds(e0 + e_l, 1), :,
                           pl.ds(wid * 4 + tc_l, 1), :, pl.ds(l0, _LH)],
                sem_w[b])

    def drain_writes(c, b):
        tc_l, e0, l0 = _chunk_coords(c)

        @pl.loop(0, _EH)
        def _d(e_l):
            pltpu.make_async_copy(
                out_stage[b].at[pl.ds(e_l, 1)],
                out_hbm.at[pl.ds(e0 + e_l, 1), :,
                           pl.ds(wid * 4 + tc_l, 1), :, pl.ds(l0, _LH)],
                sem_w[b]).wait()

    build_idx(0, 0)
    start_gather(0, 0)

    @pl.loop(0, _NCHUNK, step=2)
    def _steps(c):
        for b in (0, 1):
            cc = c + b
            nxt = cc + 1
            o = b ^ 1

            @pl.when(nxt < _NCHUNK)
            def _():
                build_idx(nxt, o)

            wait_gather(b)

            @pl.when(nxt < _NCHUNK)
            def _():
                @pl.when(cc > 0)
                def _():
                    drain_writes(cc - 1, o)
                start_gather(nxt, o)

            transpose(b)
            fire_writes(cc, b)

    drain_writes(_NCHUNK - 2, 0)
    drain_writes(_NCHUNK - 1, 1)


@jax.jit
def kernel(indices, embed):
    mesh = plsc.VectorSubcoreMesh(core_axis_name="c", subcore_axis_name="s")
    out5 = pl.kernel(
        _gather_body,
        out_type=jax.ShapeDtypeStruct((_E, _D // 8, _G // 128, 8, 128),
                                      jnp.float32),
        mesh=mesh,
        scratch_types=[
            pltpu.VMEM((_GPW, _E), jnp.int32),
            [pltpu.VMEM((_CROWS,), jnp.int32) for _ in range(2)],
            [pltpu.VMEM((_CROWS, _D), jnp.float32) for _ in range(2)],
            [pltpu.VMEM((_EH, _D // 8, 1, 8, _LH), jnp.float32)
             for _ in range(2)],
            [pltpu.SemaphoreType.DMA for _ in range(2)],
            [pltpu.SemaphoreType.DMA for _ in range(2)],
        ],
        compiler_params=pltpu.CompilerParams(use_tc_tiling_on_sc=False,
                                             needs_layout_passes=False),
    )(indices.astype(jnp.int32), embed)
    return out5.transpose(2, 4, 0, 1, 3).reshape(_G, _E, _D)
